# baseline (speedup 1.0000x reference)
import jax
import jax.numpy as jnp
from jax.experimental import pallas as pl
from jax.experimental.pallas import tpu as pltpu

_TOP_P = 0.8


def _moe_body(xb_ref, rw_ref, rb_ref, w1_ref, b1_ref, w2_ref, b2_ref,
              out_ref, gates_ref, *, n_exp):
    e = pl.program_id(1)
    c = pl.program_id(2)

    @pl.when((e == 0) & (c == 0))
    def _routing():
        logits = jnp.dot(xb_ref[...], rw_ref[...],
                         preferred_element_type=jnp.float32)
        logits = logits + rb_ref[...]
        m = jnp.max(logits, axis=-1, keepdims=True)
        ex = jnp.exp(logits - m)
        p = ex / jnp.sum(ex, axis=-1, keepdims=True)
        lane = jax.lax.broadcasted_iota(jnp.int32, p.shape, 1)
        rcols = []
        for i in range(n_exp):
            pi = p[:, i:i + 1]
            ahead = (p > pi) | ((p == pi) & (lane < i))
            rcols.append(jnp.sum(ahead.astype(jnp.int32), axis=-1,
                                 keepdims=True))
        rank = jnp.concatenate(rcols, axis=-1)
        scols = []
        for j in range(n_exp):
            sel = (rank == j).astype(p.dtype)
            scols.append(jnp.sum(p * sel, axis=-1, keepdims=True))
        acols = []
        cum = scols[0]
        acols.append(jnp.ones_like(cum))
        for j in range(1, n_exp):
            cum = cum + scols[j]
            acols.append((cum < _TOP_P).astype(p.dtype))
        active = jnp.concatenate(acols, axis=-1)
        act_val = p * active
        gcols = []
        for i in range(n_exp):
            sel = (rank[:, i:i + 1] == lane).astype(p.dtype)
            gcols.append(jnp.sum(act_val * sel, axis=-1, keepdims=True))
        gates_ref[...] = jnp.concatenate(gcols, axis=-1)

    gates = gates_ref[...]
    lane = jax.lax.broadcasted_iota(jnp.int32, gates.shape, 1)
    g = jnp.sum(gates * (lane == e).astype(gates.dtype), axis=-1,
                keepdims=True)

    @pl.when((e == 0) & (c == 0))
    def _init():
        out_ref[...] = jnp.zeros_like(out_ref)

    @pl.when(c == 0)
    def _bias2():
        out_ref[...] += g * b2_ref[0]

    h = jnp.dot(xb_ref[...], w1_ref[0],
                preferred_element_type=jnp.float32)
    h = jnp.maximum(h + b1_ref[0], 0.0)
    hg = (h * g).astype(jnp.bfloat16)
    out_ref[...] += jnp.dot(hg, w2_ref[0],
                            preferred_element_type=jnp.float32)


def kernel(x, router_w, router_b, w1, b1, w2, b2):
    B, T, H = x.shape
    E, _, I = w1.shape
    BT = B * T
    N = 512 if BT % 512 == 0 else BT
    IC = 512 if I % 512 == 0 else I
    C = I // IC

    xb = x.reshape(BT, H).astype(jnp.bfloat16)
    rwb = router_w.astype(jnp.bfloat16)
    w1b = w1.astype(jnp.bfloat16)
    w2b = w2.astype(jnp.bfloat16)
    rb2 = router_b.reshape(1, E)
    b1r = b1.reshape(E * C, 1, IC)
    b2r = b2.reshape(E, 1, H)

    grid = (BT // N, E, C)
    out = pl.pallas_call(
        lambda *refs: _moe_body(*refs, n_exp=E),
        grid=grid,
        in_specs=[
            pl.BlockSpec((N, H), lambda t, e, c: (t, 0)),
            pl.BlockSpec((H, E), lambda t, e, c: (0, 0)),
            pl.BlockSpec((1, E), lambda t, e, c: (0, 0)),
            pl.BlockSpec((1, H, IC), lambda t, e, c: (e, 0, c)),
            pl.BlockSpec((1, 1, IC), lambda t, e, c: (e * C + c, 0, 0)),
            pl.BlockSpec((1, IC, H), lambda t, e, c: (e, c, 0)),
            pl.BlockSpec((1, 1, H), lambda t, e, c: (e, 0, 0)),
        ],
        out_specs=pl.BlockSpec((N, H), lambda t, e, c: (t, 0)),
        out_shape=jax.ShapeDtypeStruct((BT, H), jnp.float32),
        scratch_shapes=[pltpu.VMEM((N, E), jnp.float32)],
    )(xb, rwb, rb2, w1b, b1r, w2b, b2r)
    return out.reshape(B, T, H)


# speedup vs baseline: 1.1830x; 1.1830x over previous
import jax
import jax.numpy as jnp
from jax.experimental import pallas as pl
from jax.experimental.pallas import tpu as pltpu

_TOP_P = 0.8


def _moe_body(xb_ref, rw_ref, rb_ref, w1_ref, b1_ref, w2_ref, b2_ref,
              out_ref, gates_ref, *, n_exp):
    e = pl.program_id(1)
    c = pl.program_id(2)

    @pl.when((e == 0) & (c == 0))
    def _routing():
        logits = jnp.dot(xb_ref[...], rw_ref[...],
                         preferred_element_type=jnp.float32)
        logits = logits + rb_ref[...]
        m = jnp.max(logits, axis=-1, keepdims=True)
        ex = jnp.exp(logits - m)
        p = ex / jnp.sum(ex, axis=-1, keepdims=True)
        lane = jax.lax.broadcasted_iota(jnp.int32, p.shape, 1)
        rcols = []
        for i in range(n_exp):
            pi = p[:, i:i + 1]
            ahead = (p > pi) | ((p == pi) & (lane < i))
            rcols.append(jnp.sum(ahead.astype(jnp.int32), axis=-1,
                                 keepdims=True))
        rank = jnp.concatenate(rcols, axis=-1)
        scols = []
        for j in range(n_exp):
            sel = (rank == j).astype(p.dtype)
            scols.append(jnp.sum(p * sel, axis=-1, keepdims=True))
        acols = []
        cum = scols[0]
        acols.append(jnp.ones_like(cum))
        for j in range(1, n_exp):
            cum = cum + scols[j]
            acols.append((cum < _TOP_P).astype(p.dtype))
        active = jnp.concatenate(acols, axis=-1)
        act_val = p * active
        gcols = []
        for i in range(n_exp):
            sel = (rank[:, i:i + 1] == lane).astype(p.dtype)
            gcols.append(jnp.sum(act_val * sel, axis=-1, keepdims=True))
        gates_ref[...] = jnp.concatenate(gcols, axis=-1)

    gates = gates_ref[...]
    lane = jax.lax.broadcasted_iota(jnp.int32, gates.shape, 1)
    g = jnp.sum(gates * (lane == e).astype(gates.dtype), axis=-1,
                keepdims=True)

    @pl.when((e == 0) & (c == 0))
    def _init():
        out_ref[...] = jnp.zeros_like(out_ref)

    @pl.when(c == 0)
    def _bias2():
        out_ref[...] += g * b2_ref[0]

    h = jnp.dot(xb_ref[...], w1_ref[0],
                preferred_element_type=jnp.float32)
    h = jnp.maximum(h + b1_ref[0], 0.0)
    hg = (h * g).astype(jnp.bfloat16)
    out_ref[...] += jnp.dot(hg, w2_ref[0],
                            preferred_element_type=jnp.float32)


def kernel(x, router_w, router_b, w1, b1, w2, b2):
    B, T, H = x.shape
    E, _, I = w1.shape
    BT = B * T
    N = 2048 if BT % 2048 == 0 else BT
    IC = 512 if I % 512 == 0 else I
    C = I // IC

    xb = x.reshape(BT, H).astype(jnp.bfloat16)
    rwb = router_w.astype(jnp.bfloat16)
    w1b = w1.astype(jnp.bfloat16)
    w2b = w2.astype(jnp.bfloat16)
    rb2 = router_b.reshape(1, E)
    b1r = b1.reshape(E * C, 1, IC)
    b2r = b2.reshape(E, 1, H)

    grid = (BT // N, E, C)
    out = pl.pallas_call(
        lambda *refs: _moe_body(*refs, n_exp=E),
        grid=grid,
        in_specs=[
            pl.BlockSpec((N, H), lambda t, e, c: (t, 0)),
            pl.BlockSpec((H, E), lambda t, e, c: (0, 0)),
            pl.BlockSpec((1, E), lambda t, e, c: (0, 0)),
            pl.BlockSpec((1, H, IC), lambda t, e, c: (e, 0, c)),
            pl.BlockSpec((1, 1, IC), lambda t, e, c: (e * C + c, 0, 0)),
            pl.BlockSpec((1, IC, H), lambda t, e, c: (e, c, 0)),
            pl.BlockSpec((1, 1, H), lambda t, e, c: (e, 0, 0)),
        ],
        out_specs=pl.BlockSpec((N, H), lambda t, e, c: (t, 0)),
        out_shape=jax.ShapeDtypeStruct((BT, H), jnp.float32),
        scratch_shapes=[pltpu.VMEM((N, E), jnp.float32)],
        compiler_params=pltpu.CompilerParams(
            dimension_semantics=("parallel", "arbitrary", "arbitrary")),
    )(xb, rwb, rb2, w1b, b1r, w2b, b2r)
    return out.reshape(B, T, H)


# speedup vs baseline: 1.2158x; 1.0277x over previous
import jax
import jax.numpy as jnp
from jax.experimental import pallas as pl
from jax.experimental.pallas import tpu as pltpu

_TOP_P = 0.8


def _moe_body(xb_ref, rw_ref, rb_ref, w1_ref, b1_ref, w2_ref, b2_ref,
              out_ref, gates_ref, *, n_exp):
    e = pl.program_id(1)
    c = pl.program_id(2)

    @pl.when((e == 0) & (c == 0))
    def _routing():
        logits = jnp.dot(xb_ref[...], rw_ref[...],
                         preferred_element_type=jnp.float32)
        logits = logits + rb_ref[...]
        m = jnp.max(logits, axis=-1, keepdims=True)
        ex = jnp.exp(logits - m)
        p = ex / jnp.sum(ex, axis=-1, keepdims=True)
        lane = jax.lax.broadcasted_iota(jnp.int32, p.shape, 1)
        rcols = []
        for i in range(n_exp):
            pi = p[:, i:i + 1]
            ahead = (p > pi) | ((p == pi) & (lane < i))
            rcols.append(jnp.sum(ahead.astype(jnp.int32), axis=-1,
                                 keepdims=True))
        rank = jnp.concatenate(rcols, axis=-1)
        scols = []
        for j in range(n_exp):
            sel = (rank == j).astype(p.dtype)
            scols.append(jnp.sum(p * sel, axis=-1, keepdims=True))
        acols = []
        cum = scols[0]
        acols.append(jnp.ones_like(cum))
        for j in range(1, n_exp):
            cum = cum + scols[j]
            acols.append((cum < _TOP_P).astype(p.dtype))
        active = jnp.concatenate(acols, axis=-1)
        act_val = p * active
        for i in range(n_exp):
            sel = (rank[:, i:i + 1] == lane).astype(p.dtype)
            gates_ref[i] = jnp.sum(act_val * sel, axis=-1, keepdims=True)

    g = gates_ref[e]

    @pl.when((e == 0) & (c == 0))
    def _init():
        out_ref[...] = jnp.zeros_like(out_ref)

    @pl.when(c == 0)
    def _bias2():
        out_ref[...] += g * b2_ref[0]

    h = jnp.dot(xb_ref[...], w1_ref[0],
                preferred_element_type=jnp.float32)
    h = jnp.maximum(h + b1_ref[0], 0.0)
    hg = (h * g).astype(jnp.bfloat16)
    out_ref[...] += jnp.dot(hg, w2_ref[0],
                            preferred_element_type=jnp.float32)


def kernel(x, router_w, router_b, w1, b1, w2, b2):
    B, T, H = x.shape
    E, _, I = w1.shape
    BT = B * T
    N = 2048 if BT % 2048 == 0 else BT
    IC = 512 if I % 512 == 0 else I
    C = I // IC

    xb = x.reshape(BT, H).astype(jnp.bfloat16)
    rwb = router_w.astype(jnp.bfloat16)
    w1b = w1.astype(jnp.bfloat16)
    w2b = w2.astype(jnp.bfloat16)
    rb2 = router_b.reshape(1, E)
    b1r = b1.reshape(E * C, 1, IC)
    b2r = b2.reshape(E, 1, H)

    grid = (BT // N, E, C)
    out = pl.pallas_call(
        lambda *refs: _moe_body(*refs, n_exp=E),
        grid=grid,
        in_specs=[
            pl.BlockSpec((N, H), lambda t, e, c: (t, 0)),
            pl.BlockSpec((H, E), lambda t, e, c: (0, 0)),
            pl.BlockSpec((1, E), lambda t, e, c: (0, 0)),
            pl.BlockSpec((1, H, IC), lambda t, e, c: (e, 0, c)),
            pl.BlockSpec((1, 1, IC), lambda t, e, c: (e * C + c, 0, 0)),
            pl.BlockSpec((1, IC, H), lambda t, e, c: (e, c, 0)),
            pl.BlockSpec((1, 1, H), lambda t, e, c: (e, 0, 0)),
        ],
        out_specs=pl.BlockSpec((N, H), lambda t, e, c: (t, 0)),
        out_shape=jax.ShapeDtypeStruct((BT, H), jnp.float32),
        scratch_shapes=[pltpu.VMEM((E, N, 1), jnp.float32)],
        compiler_params=pltpu.CompilerParams(
            dimension_semantics=("parallel", "arbitrary", "arbitrary")),
    )(xb, rwb, rb2, w1b, b1r, w2b, b2r)
    return out.reshape(B, T, H)


# speedup vs baseline: 1.2514x; 1.0293x over previous
import jax
import jax.numpy as jnp
from jax.experimental import pallas as pl
from jax.experimental.pallas import tpu as pltpu

_TOP_P = 0.8


def _routing(xb, rw, rb, gates_ref, n_exp):
    logits = jnp.dot(xb, rw, preferred_element_type=jnp.float32) + rb
    m = jnp.max(logits, axis=-1, keepdims=True)
    ex = jnp.exp(logits - m)
    p = ex / jnp.sum(ex, axis=-1, keepdims=True)
    lane = jax.lax.broadcasted_iota(jnp.int32, p.shape, 1)
    rcols = []
    for i in range(n_exp):
        pi = p[:, i:i + 1]
        ahead = (p > pi) | ((p == pi) & (lane < i))
        rcols.append(jnp.sum(ahead.astype(jnp.int32), axis=-1, keepdims=True))
    rank = jnp.concatenate(rcols, axis=-1)
    scols = []
    for j in range(n_exp):
        sel = (rank == j).astype(p.dtype)
        scols.append(jnp.sum(p * sel, axis=-1, keepdims=True))
    acols = [jnp.ones_like(scols[0])]
    cum = scols[0]
    for j in range(1, n_exp):
        cum = cum + scols[j]
        acols.append((cum < _TOP_P).astype(p.dtype))
    active = jnp.concatenate(acols, axis=-1)
    act_val = p * active
    for i in range(n_exp):
        sel = (rank[:, i:i + 1] == lane).astype(p.dtype)
        gates_ref[i] = jnp.sum(act_val * sel, axis=-1, keepdims=True)


def _moe_body(xb_ref, rw_ref, rb_ref, w1_ref, b1_ref, w2_ref, b2_ref,
              out_ref, gates_ref, *, n_exp, n_chunks):
    e = pl.program_id(1)

    @pl.when(e == 0)
    def _route():
        _routing(xb_ref[...], rw_ref[...], rb_ref[...], gates_ref, n_exp)

    g = gates_ref[e]
    n = out_ref.shape[0]
    ck = n // n_chunks
    for s in range(n_chunks):
        lo, hi = s * ck, (s + 1) * ck
        gs = g[lo:hi]
        h = jnp.dot(xb_ref[lo:hi, :], w1_ref[0],
                    preferred_element_type=jnp.float32)
        h = jnp.maximum(h + b1_ref[0], 0.0)
        hg = (h * gs).astype(jnp.bfloat16)
        contrib = jnp.dot(hg, w2_ref[0],
                          preferred_element_type=jnp.float32)
        contrib = contrib + gs * b2_ref[0]

        @pl.when(e == 0)
        def _set():
            out_ref[lo:hi, :] = contrib

        @pl.when(e > 0)
        def _acc():
            out_ref[lo:hi, :] += contrib


def kernel(x, router_w, router_b, w1, b1, w2, b2):
    B, T, H = x.shape
    E, _, I = w1.shape
    BT = B * T
    N = 2048 if BT % 2048 == 0 else BT
    n_chunks = 4 if N % (4 * 256) == 0 else 1

    xb = x.reshape(BT, H).astype(jnp.bfloat16)
    rwb = router_w.astype(jnp.bfloat16)
    w1b = w1.astype(jnp.bfloat16)
    w2b = w2.astype(jnp.bfloat16)
    rb2 = router_b.reshape(1, E)
    b1r = b1.reshape(E, 1, I)
    b2r = b2.reshape(E, 1, H)

    grid = (BT // N, E)
    out = pl.pallas_call(
        lambda *refs: _moe_body(*refs, n_exp=E, n_chunks=n_chunks),
        grid=grid,
        in_specs=[
            pl.BlockSpec((N, H), lambda t, e: (t, 0)),
            pl.BlockSpec((H, E), lambda t, e: (0, 0)),
            pl.BlockSpec((1, E), lambda t, e: (0, 0)),
            pl.BlockSpec((1, H, I), lambda t, e: (e, 0, 0)),
            pl.BlockSpec((1, 1, I), lambda t, e: (e, 0, 0)),
            pl.BlockSpec((1, I, H), lambda t, e: (e, 0, 0)),
            pl.BlockSpec((1, 1, H), lambda t, e: (e, 0, 0)),
        ],
        out_specs=pl.BlockSpec((N, H), lambda t, e: (t, 0)),
        out_shape=jax.ShapeDtypeStruct((BT, H), jnp.float32),
        scratch_shapes=[pltpu.VMEM((E, N, 1), jnp.float32)],
        compiler_params=pltpu.CompilerParams(
            dimension_semantics=("parallel", "arbitrary")),
    )(xb, rwb, rb2, w1b, b1r, w2b, b2r)
    return out.reshape(B, T, H)


# speedup vs baseline: 1.3094x; 1.0464x over previous
import jax
import jax.numpy as jnp
from jax.experimental import pallas as pl
from jax.experimental.pallas import tpu as pltpu

_TOP_P = 0.8


def _routing(xb, rw, rb, gates_ref, n_exp):
    logits = jnp.dot(xb, rw, preferred_element_type=jnp.float32) + rb
    m = jnp.max(logits, axis=-1, keepdims=True)
    ex = jnp.exp(logits - m)
    p = ex / jnp.sum(ex, axis=-1, keepdims=True)
    lane = jax.lax.broadcasted_iota(jnp.int32, p.shape, 1)
    rcols = []
    for i in range(n_exp):
        pi = p[:, i:i + 1]
        ahead = (p > pi) | ((p == pi) & (lane < i))
        rcols.append(jnp.sum(ahead.astype(jnp.int32), axis=-1, keepdims=True))
    rank = jnp.concatenate(rcols, axis=-1)
    scols = []
    for j in range(n_exp):
        sel = (rank == j).astype(p.dtype)
        scols.append(jnp.sum(p * sel, axis=-1, keepdims=True))
    acols = [jnp.ones_like(scols[0])]
    cum = scols[0]
    for j in range(1, n_exp):
        cum = cum + scols[j]
        acols.append((cum < _TOP_P).astype(p.dtype))
    active = jnp.concatenate(acols, axis=-1)
    act_val = p * active
    for i in range(n_exp):
        sel = (rank[:, i:i + 1] == lane).astype(p.dtype)
        gates_ref[i] = jnp.sum(act_val * sel, axis=-1, keepdims=True)


def _moe_body(xb_ref, rw_ref, rb_ref, w1_ref, b1_ref, w2_ref, b2_ref,
              out_ref, gates_ref, *, n_exp, n_chunks):
    e = pl.program_id(1)

    @pl.when(e == 0)
    def _route():
        _routing(xb_ref[...], rw_ref[...], rb_ref[...], gates_ref, n_exp)

    g = gates_ref[e]
    n = out_ref.shape[0]
    ck = n // n_chunks
    keep = (e > 0).astype(jnp.float32)
    for s in range(n_chunks):
        lo, hi = s * ck, (s + 1) * ck
        gs = g[lo:hi]
        h = jnp.dot(xb_ref[lo:hi, :], w1_ref[0],
                    preferred_element_type=jnp.float32)
        h = jnp.maximum(h + b1_ref[0], 0.0)
        hg = (h * gs).astype(jnp.bfloat16)
        contrib = jnp.dot(hg, w2_ref[0],
                          preferred_element_type=jnp.float32)
        contrib = contrib + gs * b2_ref[0]
        out_ref[lo:hi, :] = out_ref[lo:hi, :] * keep + contrib


def kernel(x, router_w, router_b, w1, b1, w2, b2):
    B, T, H = x.shape
    E, _, I = w1.shape
    BT = B * T
    N = 2048 if BT % 2048 == 0 else BT
    n_chunks = 4 if N % (4 * 256) == 0 else 1

    xb = x.reshape(BT, H).astype(jnp.bfloat16)
    rwb = router_w.astype(jnp.bfloat16)
    w1b = w1.astype(jnp.bfloat16)
    w2b = w2.astype(jnp.bfloat16)
    rb2 = router_b.reshape(1, E)
    b1r = b1.reshape(E, 1, I)
    b2r = b2.reshape(E, 1, H)

    grid = (BT // N, E)
    out = pl.pallas_call(
        lambda *refs: _moe_body(*refs, n_exp=E, n_chunks=n_chunks),
        grid=grid,
        in_specs=[
            pl.BlockSpec((N, H), lambda t, e: (t, 0)),
            pl.BlockSpec((H, E), lambda t, e: (0, 0)),
            pl.BlockSpec((1, E), lambda t, e: (0, 0)),
            pl.BlockSpec((1, H, I), lambda t, e: (e, 0, 0)),
            pl.BlockSpec((1, 1, I), lambda t, e: (e, 0, 0)),
            pl.BlockSpec((1, I, H), lambda t, e: (e, 0, 0)),
            pl.BlockSpec((1, 1, H), lambda t, e: (e, 0, 0)),
        ],
        out_specs=pl.BlockSpec((N, H), lambda t, e: (t, 0)),
        out_shape=jax.ShapeDtypeStruct((BT, H), jnp.float32),
        scratch_shapes=[pltpu.VMEM((E, N, 1), jnp.float32)],
        compiler_params=pltpu.CompilerParams(
            dimension_semantics=("parallel", "arbitrary")),
    )(xb, rwb, rb2, w1b, b1r, w2b, b2r)
    return out.reshape(B, T, H)


# speedup vs baseline: 1.3192x; 1.0074x over previous
import jax
import jax.numpy as jnp
from jax.experimental import pallas as pl
from jax.experimental.pallas import tpu as pltpu

_TOP_P = 0.8


def _routing(xb, rw, rb, gates_ref, n_exp):
    logits = jnp.dot(xb, rw, preferred_element_type=jnp.float32) + rb
    m = jnp.max(logits, axis=-1, keepdims=True)
    ex = jnp.exp(logits - m)
    p = ex / jnp.sum(ex, axis=-1, keepdims=True)
    lane = jax.lax.broadcasted_iota(jnp.int32, p.shape, 1)
    rcols = []
    for i in range(n_exp):
        pi = p[:, i:i + 1]
        ahead = (p > pi) | ((p == pi) & (lane < i))
        rcols.append(jnp.sum(ahead.astype(jnp.int32), axis=-1, keepdims=True))
    rank = jnp.concatenate(rcols, axis=-1)
    scols = []
    for j in range(n_exp):
        sel = (rank == j).astype(p.dtype)
        scols.append(jnp.sum(p * sel, axis=-1, keepdims=True))
    acols = [jnp.ones_like(scols[0])]
    cum = scols[0]
    for j in range(1, n_exp):
        cum = cum + scols[j]
        acols.append((cum < _TOP_P).astype(p.dtype))
    active = jnp.concatenate(acols, axis=-1)
    act_val = p * active
    for i in range(n_exp):
        sel = (rank[:, i:i + 1] == lane).astype(p.dtype)
        gates_ref[i] = jnp.sum(act_val * sel, axis=-1, keepdims=True)


def _moe_body(xb_ref, rw_ref, rb_ref, w1_ref, b1_ref, w2_ref, b2_ref,
              out_ref, gates_ref, *, n_exp, n_chunks):
    e = pl.program_id(1)

    @pl.when(e == 0)
    def _route():
        _routing(xb_ref[...], rw_ref[...], rb_ref[...], gates_ref, n_exp)

    g = gates_ref[e]
    n = out_ref.shape[0]
    ck = n // n_chunks
    keep = e > 0
    for s in range(n_chunks):
        lo, hi = s * ck, (s + 1) * ck
        gs = g[lo:hi]
        h = jnp.dot(xb_ref[lo:hi, :], w1_ref[0],
                    preferred_element_type=jnp.float32)
        h = jnp.maximum(h + b1_ref[0], 0.0)
        hg = (h * gs).astype(jnp.bfloat16)
        contrib = jnp.dot(hg, w2_ref[0],
                          preferred_element_type=jnp.float32)
        contrib = contrib + gs * b2_ref[0]
        prev = jnp.where(keep, out_ref[lo:hi, :], 0.0)
        out_ref[lo:hi, :] = prev + contrib


def kernel(x, router_w, router_b, w1, b1, w2, b2):
    B, T, H = x.shape
    E, _, I = w1.shape
    BT = B * T
    N = 2048 if BT % 2048 == 0 else BT
    n_chunks = 4 if N % (4 * 256) == 0 else 1

    xb = x.reshape(BT, H).astype(jnp.bfloat16)
    rwb = router_w.astype(jnp.bfloat16)
    w1b = w1.astype(jnp.bfloat16)
    w2b = w2.astype(jnp.bfloat16)
    rb2 = router_b.reshape(1, E)
    b1r = b1.reshape(E, 1, I)
    b2r = b2.reshape(E, 1, H)

    grid = (BT // N, E)
    out = pl.pallas_call(
        lambda *refs: _moe_body(*refs, n_exp=E, n_chunks=n_chunks),
        grid=grid,
        in_specs=[
            pl.BlockSpec((N, H), lambda t, e: (t, 0)),
            pl.BlockSpec((H, E), lambda t, e: (0, 0)),
            pl.BlockSpec((1, E), lambda t, e: (0, 0)),
            pl.BlockSpec((1, H, I), lambda t, e: (e, 0, 0)),
            pl.BlockSpec((1, 1, I), lambda t, e: (e, 0, 0)),
            pl.BlockSpec((1, I, H), lambda t, e: (e, 0, 0)),
            pl.BlockSpec((1, 1, H), lambda t, e: (e, 0, 0)),
        ],
        out_specs=pl.BlockSpec((N, H), lambda t, e: (t, 0)),
        out_shape=jax.ShapeDtypeStruct((BT, H), jnp.float32),
        scratch_shapes=[pltpu.VMEM((E, N, 1), jnp.float32)],
        compiler_params=pltpu.CompilerParams(
            dimension_semantics=("parallel", "arbitrary")),
    )(xb, rwb, rb2, w1b, b1r, w2b, b2r)
    return out.reshape(B, T, H)


# speedup vs baseline: 1.3213x; 1.0016x over previous
import jax
import jax.numpy as jnp
from jax.experimental import pallas as pl
from jax.experimental.pallas import tpu as pltpu

_TOP_P = 0.8


def _routing(xb, rw, rb, gates_ref, n_exp):
    logits = jnp.dot(xb, rw, preferred_element_type=jnp.float32) + rb
    m = jnp.max(logits, axis=-1, keepdims=True)
    ex = jnp.exp(logits - m)
    p = ex / jnp.sum(ex, axis=-1, keepdims=True)
    lane = jax.lax.broadcasted_iota(jnp.int32, p.shape, 1)
    rcols = []
    for i in range(n_exp):
        pi = p[:, i:i + 1]
        ahead = (p > pi) | ((p == pi) & (lane < i))
        rcols.append(jnp.sum(ahead.astype(jnp.int32), axis=-1, keepdims=True))
    rank = jnp.concatenate(rcols, axis=-1)
    scols = []
    for j in range(n_exp):
        sel = (rank == j).astype(p.dtype)
        scols.append(jnp.sum(p * sel, axis=-1, keepdims=True))
    acols = [jnp.ones_like(scols[0])]
    cum = scols[0]
    for j in range(1, n_exp):
        cum = cum + scols[j]
        acols.append((cum < _TOP_P).astype(p.dtype))
    active = jnp.concatenate(acols, axis=-1)
    act_val = p * active
    for i in range(n_exp):
        sel = (rank[:, i:i + 1] == lane).astype(p.dtype)
        gates_ref[i] = jnp.sum(act_val * sel, axis=-1, keepdims=True)


def _moe_body(xb_ref, rw_ref, rb_ref, w1_ref, b1_ref, w2_ref, b2_ref,
              out_ref, gates_ref, *, n_exp, n_chunks):
    e = pl.program_id(1)

    @pl.when(e == 0)
    def _route():
        _routing(xb_ref[...], rw_ref[...], rb_ref[...], gates_ref, n_exp)

    g = gates_ref[e]
    n = out_ref.shape[0]
    ck = n // n_chunks
    keep = e > 0
    for s in range(n_chunks):
        lo, hi = s * ck, (s + 1) * ck
        gs = g[lo:hi]
        h = jnp.dot(xb_ref[lo:hi, :], w1_ref[0],
                    preferred_element_type=jnp.float32)
        h = jnp.maximum(h + b1_ref[0], 0.0).astype(jnp.bfloat16)
        contrib = jnp.dot(h, w2_ref[0],
                          preferred_element_type=jnp.float32)
        contrib = (contrib + b2_ref[0]) * gs
        prev = jnp.where(keep, out_ref[lo:hi, :], 0.0)
        out_ref[lo:hi, :] = prev + contrib


def kernel(x, router_w, router_b, w1, b1, w2, b2):
    B, T, H = x.shape
    E, _, I = w1.shape
    BT = B * T
    N = 2048 if BT % 2048 == 0 else BT
    n_chunks = 4 if N % (4 * 256) == 0 else 1

    xb = x.reshape(BT, H).astype(jnp.bfloat16)
    rwb = router_w.astype(jnp.bfloat16)
    w1b = w1.astype(jnp.bfloat16)
    w2b = w2.astype(jnp.bfloat16)
    rb2 = router_b.reshape(1, E)
    b1r = b1.reshape(E, 1, I)
    b2r = b2.reshape(E, 1, H)

    grid = (BT // N, E)
    out = pl.pallas_call(
        lambda *refs: _moe_body(*refs, n_exp=E, n_chunks=n_chunks),
        grid=grid,
        in_specs=[
            pl.BlockSpec((N, H), lambda t, e: (t, 0)),
            pl.BlockSpec((H, E), lambda t, e: (0, 0)),
            pl.BlockSpec((1, E), lambda t, e: (0, 0)),
            pl.BlockSpec((1, H, I), lambda t, e: (e, 0, 0)),
            pl.BlockSpec((1, 1, I), lambda t, e: (e, 0, 0)),
            pl.BlockSpec((1, I, H), lambda t, e: (e, 0, 0)),
            pl.BlockSpec((1, 1, H), lambda t, e: (e, 0, 0)),
        ],
        out_specs=pl.BlockSpec((N, H), lambda t, e: (t, 0)),
        out_shape=jax.ShapeDtypeStruct((BT, H), jnp.float32),
        scratch_shapes=[pltpu.VMEM((E, N, 1), jnp.float32)],
        compiler_params=pltpu.CompilerParams(
            dimension_semantics=("parallel", "arbitrary")),
    )(xb, rwb, rb2, w1b, b1r, w2b, b2r)
    return out.reshape(B, T, H)
